# Initial kernel scaffold; baseline (speedup 1.0000x reference)
import jax
import jax.numpy as jnp
from jax.experimental import pallas as pl


def kernel(tokens, table):
    raise NotImplementedError("write your pallas kernel here")


# speedup vs baseline: 1.1315x; 1.1315x over previous
import functools

import numpy as np
import jax
import jax.numpy as jnp
from jax import lax
from jax.experimental import pallas as pl
from jax.experimental.pallas import tpu as pltpu
from jax.experimental.pallas import tpu_sc as plsc

_VOCAB = 100000
_D = 768
_S = 2048
_B = 32
_NC = 2
_NS = 16
_NW = _NC * _NS
_PCHUNK = _S // _NW
_LANES = 16
_CVEC = _D // _LANES


def _pe_table() -> np.ndarray:
    even_i = np.arange(0, _D, 2, dtype=np.float32)
    denominator = np.power(np.float32(10000.0), even_i / np.float32(_D))
    position = np.arange(_S, dtype=np.float32).reshape(_S, 1)
    even_pe = np.sin(position / denominator)
    odd_pe = np.cos(position / denominator)
    pe = np.stack([even_pe, odd_pe], axis=2).reshape(_S, _D)
    return pe.astype(np.float32)


_PE = _pe_table()

_MESH = plsc.VectorSubcoreMesh(core_axis_name="c", subcore_axis_name="s")


@functools.partial(
    pl.kernel,
    out_type=jax.ShapeDtypeStruct((_B, _S, _D), jnp.float32),
    mesh=_MESH,
    scratch_types=[
        pltpu.VMEM((_PCHUNK,), jnp.int32),
        pltpu.VMEM((_PCHUNK, _D), jnp.float32),
        pltpu.VMEM((_PCHUNK, _D), jnp.float32),
        pltpu.SemaphoreType.DMA,
    ],
)
def _embed(tokens_hbm, table_hbm, pe_hbm, out_hbm, idx_v, pe_v, rows_v, sem):
    wid = lax.axis_index("s") * _NC + lax.axis_index("c")
    p0 = wid * _PCHUNK
    pltpu.sync_copy(pe_hbm.at[pl.ds(p0, _PCHUNK)], pe_v)

    def batch_body(b, carry):
        pltpu.sync_copy(tokens_hbm.at[b, pl.ds(p0, _PCHUNK)], idx_v)
        pltpu.async_copy(table_hbm.at[idx_v], rows_v, sem).wait()

        def row_body(r, c2):
            for c in range(_CVEC):
                sl = pl.ds(c * _LANES, _LANES)
                plsc.addupdate(rows_v.at[r, sl], pe_v[r, sl])
            return c2

        lax.fori_loop(0, _PCHUNK, row_body, 0)
        pltpu.sync_copy(rows_v, out_hbm.at[b, pl.ds(p0, _PCHUNK)])
        return carry

    lax.fori_loop(0, _B, batch_body, 0)


def kernel(tokens, table):
    return _embed(tokens, table, jnp.asarray(_PE))
